# baseline (speedup 1.0000x reference)
import jax
import jax.numpy as jnp
import numpy as np
from jax import lax
from jax.experimental import pallas as pl
from jax.experimental.pallas import tpu as pltpu
from jax.experimental.pallas import tpu_sc as plsc

_DEPTH = 3
_STEP = 0.5
_AB = [(0.0, float(b)) for b in np.arange(0, 3, _STEP)] + [
    (float(a), 0.0) for a in np.arange(_STEP, 3, _STEP)
]
_M = len(_AB)
_N = 10000
_E = 320000
_D = 128
_DH = _D // 2

_NT = 16
_NPAD = 10240
_RT = _NPAD // _NT
_ET = _E // _NT
_BE = 128
_NB = 158
_ETP = _NB * _BE

_f32 = jnp.float32
_i32 = jnp.int32


def _coef_matrix(alphas, w):
    a = jnp.array([t[0] for t in _AB], _f32)
    b = jnp.array([t[1] for t in _AB], _f32)
    c0 = jnp.zeros((_M, _DEPTH + 1), _f32).at[:, 0].set(1.0)
    al0 = alphas[0]
    c1 = (
        jnp.zeros((_M, _DEPTH + 1), _f32)
        .at[:, 0].set(al0 * (a - b) / 2.0)
        .at[:, 1].set(al0 * (a + b + 2.0) / 2.0)
    )
    cs = [c0, c1]
    for L in range(2, _DEPTH + 1):
        coef_l = 2.0 * L * (L + a + b) * (2.0 * L - 2.0 + a + b)
        coef_lm1_1 = (2.0 * L + a + b - 1.0) * (2.0 * L + a + b) * (2.0 * L + a + b - 2.0)
        coef_lm1_2 = (2.0 * L + a + b - 1.0) * (a * a - b * b)
        coef_lm2 = 2.0 * (L - 1.0 + a) * (L - 1.0 + b) * (2.0 * L + a + b)
        tmp1 = alphas[L - 1] * (coef_lm1_1 / coef_l)
        tmp2 = alphas[L - 1] * (coef_lm1_2 / coef_l)
        tmp3 = alphas[L - 1] * alphas[L - 2] * (coef_lm2 / coef_l)
        cp, cpp = cs[L - 1], cs[L - 2]
        shifted = jnp.concatenate([jnp.zeros((_M, 1), _f32), cp[:, :_DEPTH]], axis=1)
        cs.append(tmp1[:, None] * shifted - tmp2[:, None] * cp - tmp3[:, None] * cpp)
    wm = w.reshape(_M)
    return jnp.stack([(wm[:, None] * cL).sum(0) for cL in cs]).reshape(-1)


def _rsqrt16(d):
    i = lax.bitcast_convert_type(d, _i32)
    i = jnp.int32(0x5F3759DF) - lax.shift_right_arithmetic(i, jnp.int32(1))
    y = lax.bitcast_convert_type(i, _f32)
    for _ in range(3):
        y = y * (1.5 - 0.5 * d * y * y)
    return y


def _sc_body(xt_r, rowt_r, colt_r, attrt_r, cmat_r,
             out_r, y1_r, y2_r, y3_r,
             row_v, col_v, val_v, dinv_v, gbufA, gbufB,
             cvec, deg_sh, acc_sh, semA, semB, semSA, semSB):
    cid = lax.axis_index("c")
    sid = lax.axis_index("s")
    z16 = jnp.zeros((16,), _f32)
    one16 = jnp.ones((16,), _f32)

    pltpu.sync_copy(rowt_r.at[sid], row_v)
    pltpu.sync_copy(colt_r.at[sid], col_v)
    pltpu.sync_copy(attrt_r.at[sid], val_v)
    pltpu.sync_copy(cmat_r, cvec)

    @pl.loop(0, _BE)
    def _zero_rows0(r):
        for q in range(_DH // 16):
            gbufA[r, pl.ds(q * 16, 16)] = z16
    for k in range(_RT // _BE):
        pltpu.sync_copy(gbufA, acc_sh.at[pl.ds(sid * _RT + k * _BE, _BE)])

    @pl.loop(0, _BE)
    def _fill_ones(r):
        for q in range(_DH // 16):
            gbufA[r, pl.ds(q * 16, 16)] = one16
    plsc.subcore_barrier()

    @pl.loop(0, _NB // 2)
    def _hist(p):
        b = p * 2
        pltpu.async_copy(gbufA, acc_sh.at[row_v.at[b]], semSA, add=True)
        pltpu.async_copy(gbufA, acc_sh.at[row_v.at[b + 1]], semSB, add=True)
        pltpu.make_async_copy(gbufA, acc_sh.at[pl.ds(0, _BE)], semSA).wait()
        pltpu.make_async_copy(gbufA, acc_sh.at[pl.ds(0, _BE)], semSB).wait()
    plsc.subcore_barrier()

    iota16 = lax.iota(_i32, 16)
    for k in range(_RT // _BE):
        pltpu.sync_copy(acc_sh.at[pl.ds(sid * _RT + k * _BE, _BE)], gbufA)
        for g in range(_BE // 16):
            d = plsc.load_gather(gbufA, [iota16 + g * 16, jnp.zeros((16,), _i32)])
            d = d + jnp.where(d < 0.5, 1.0, 0.0).astype(_f32)
            dinv_v[pl.ds(k * _BE + g * 16, 16)] = _rsqrt16(d)
    pltpu.sync_copy(dinv_v.at[pl.ds(0, _RT)], deg_sh.at[pl.ds(sid * _RT, _RT)])
    plsc.subcore_barrier()

    pltpu.sync_copy(deg_sh, dinv_v)

    @pl.loop(0, _NB)
    def _val(bi):
        for g in range(_BE // 16):
            sl = pl.ds(g * 16, 16)
            r16 = row_v[bi, sl]
            c16 = col_v[bi, sl]
            dr = plsc.load_gather(dinv_v, [r16])
            dc = plsc.load_gather(dinv_v, [c16])
            val_v[bi, sl] = dr * val_v[bi, sl] * dc

    off16 = jnp.full((16,), cid * _NPAD, _i32)

    @pl.loop(0, _NB)
    def _off(bi):
        for g in range(_BE // 16):
            sl = pl.ds(g * 16, 16)
            col_v[bi, sl] = col_v[bi, sl] + off16

    def _scale(buf, bi):
        @pl.loop(0, _BE // 16)
        def _grp(g):
            val16 = val_v[bi, pl.ds(g * 16, 16)]
            e0 = g * 16
            for i in range(16):
                vb = lax.gather(
                    val16, jnp.full((16, 1), i, _i32),
                    lax.GatherDimensionNumbers(
                        offset_dims=(), collapsed_slice_dims=(0,),
                        start_index_map=(0,)),
                    slice_sizes=(1,),
                    mode=lax.GatherScatterMode.PROMISE_IN_BOUNDS)
                for q in range(_DH // 16):
                    sl = pl.ds(q * 16, 16)
                    buf[e0 + i, sl] = buf[e0 + i, sl] * vb

    for src, dst in ((xt_r, y1_r), (y1_r, y2_r), (y2_r, y3_r)):
        @pl.loop(0, _BE)
        def _zero_rows(r):
            for q in range(_DH // 16):
                gbufA[r, pl.ds(q * 16, 16)] = z16
        for k in range(_RT // _BE):
            pltpu.sync_copy(gbufA, acc_sh.at[pl.ds(sid * _RT + k * _BE, _BE)])
        plsc.subcore_barrier()

        pltpu.async_copy(src.at[col_v.at[0]], gbufA, semA)

        @pl.loop(0, _NB // 2)
        def _pairs(p, src=src):
            b = p * 2
            pltpu.async_copy(src.at[col_v.at[b + 1]], gbufB, semB)
            pltpu.make_async_copy(src.at[pl.ds(0, _BE)], gbufA, semA).wait()
            _scale(gbufA, b)
            pltpu.async_copy(gbufA, acc_sh.at[row_v.at[b]], semSA, add=True)
            pltpu.make_async_copy(src.at[pl.ds(0, _BE)], gbufB, semB).wait()
            _scale(gbufB, b + 1)
            pltpu.async_copy(gbufB, acc_sh.at[row_v.at[b + 1]], semSB, add=True)
            pltpu.make_async_copy(gbufA, acc_sh.at[pl.ds(0, _BE)], semSA).wait()

            @pl.when(b + 2 < _NB)
            def _fire_next():
                pltpu.async_copy(src.at[col_v.at[b + 2]], gbufA, semA)

            pltpu.make_async_copy(gbufB, acc_sh.at[pl.ds(0, _BE)], semSB).wait()

        plsc.subcore_barrier()
        for k in range(_RT // _BE):
            r0 = sid * _RT + k * _BE
            pltpu.sync_copy(acc_sh.at[pl.ds(r0, _BE)], dst.at[pl.ds(cid * _NPAD + r0, _BE)])
        plsc.subcore_barrier()

    cb = [[cvec[4 * L + j, pl.ds(0, 16)] for j in range(4)] for L in range(4)]
    srcs = (xt_r, y1_r, y2_r, y3_r)

    @pl.loop(0, _RT // 32)
    def _combine(k):
        r0 = sid * _RT + k * 32
        for j in range(4):
            pltpu.async_copy(srcs[j].at[pl.ds(cid * _NPAD + r0, 32)],
                             gbufA.at[pl.ds(32 * j, 32)], semA)
        for j in range(4):
            pltpu.make_async_copy(srcs[0].at[pl.ds(0, 32)],
                                  gbufA.at[pl.ds(0, 32)], semA).wait()

        @pl.loop(0, 32)
        def _rows(r):
            for q in range(_DH // 16):
                sl = pl.ds(q * 16, 16)
                a = [gbufA[32 * j + r, sl] for j in range(4)]
                for L in range(4):
                    gbufB[r * 4 + L, sl] = (
                        a[0] * cb[L][0] + a[1] * cb[L][1]
                        + a[2] * cb[L][2] + a[3] * cb[L][3]
                    )

        pltpu.sync_copy(gbufB, out_r.at[cid, pl.ds(r0 * 4, _BE)])


@jax.jit
def kernel(x, edge_index, edge_attr, alphas, w):
    cmat = jnp.tile(_coef_matrix(alphas, w).reshape(16, 1), (1, 16))

    xp = jnp.pad(x, ((0, _NPAD - _N), (0, 0)))
    xt = xp.reshape(_NPAD, 2, _DH).transpose(1, 0, 2).reshape(2 * _NPAD, _DH)

    row = jnp.pad(edge_index[0].reshape(_NT, _ET), ((0, 0), (0, _ETP - _ET)),
                  constant_values=_N).reshape(_NT, _NB, _BE)
    col = jnp.pad(edge_index[1].reshape(_NT, _ET), ((0, 0), (0, _ETP - _ET)),
                  constant_values=0).reshape(_NT, _NB, _BE)
    attr = jnp.pad(edge_attr.reshape(_NT, _ET), ((0, 0), (0, _ETP - _ET)),
                   constant_values=0.0).reshape(_NT, _NB, _BE)

    mesh = plsc.VectorSubcoreMesh(core_axis_name="c", subcore_axis_name="s")
    fn = pl.kernel(
        _sc_body,
        out_type=(
            jax.ShapeDtypeStruct((2, _NPAD * (_DEPTH + 1), _DH), _f32),
            jax.ShapeDtypeStruct((2 * _NPAD, _DH), _f32),
            jax.ShapeDtypeStruct((2 * _NPAD, _DH), _f32),
            jax.ShapeDtypeStruct((2 * _NPAD, _DH), _f32),
        ),
        mesh=mesh,
        compiler_params=pltpu.CompilerParams(
            needs_layout_passes=False, use_tc_tiling_on_sc=False),
        scratch_types=[
            pltpu.VMEM((_NB, _BE), _i32),
            pltpu.VMEM((_NB, _BE), _i32),
            pltpu.VMEM((_NB, _BE), _f32),
            pltpu.VMEM((_NPAD,), _f32),
            pltpu.VMEM((_BE, _DH), _f32),
            pltpu.VMEM((_BE, _DH), _f32),
            pltpu.VMEM((16, 16), _f32),
            pltpu.VMEM_SHARED((_NPAD,), _f32),
            pltpu.VMEM_SHARED((_NPAD, _DH), _f32),
            pltpu.SemaphoreType.DMA,
            pltpu.SemaphoreType.DMA,
            pltpu.SemaphoreType.DMA,
            pltpu.SemaphoreType.DMA,
        ],
    )
    out_t, _, _, _ = fn(xt, row, col, attr, cmat)
    return (out_t.reshape(2, _NPAD, _DEPTH + 1, _DH)[:, :_N]
            .transpose(1, 2, 0, 3).reshape(_N, _DEPTH + 1, _D))


# speedup vs baseline: 1.2168x; 1.2168x over previous
import jax
import jax.numpy as jnp
import numpy as np
from jax import lax
from jax.experimental import pallas as pl
from jax.experimental.pallas import tpu as pltpu
from jax.experimental.pallas import tpu_sc as plsc

_DEPTH = 3
_STEP = 0.5
_AB = [(0.0, float(b)) for b in np.arange(0, 3, _STEP)] + [
    (float(a), 0.0) for a in np.arange(_STEP, 3, _STEP)
]
_M = len(_AB)
_N = 10000
_E = 320000
_D = 128
_DH = _D // 2

_NT = 16
_NPAD = 10240
_RT = _NPAD // _NT
_ET = _E // _NT
_BE = 128
_NB = 158
_ETP = _NB * _BE

_f32 = jnp.float32
_i32 = jnp.int32


def _coef_matrix(alphas, w):
    a = jnp.array([t[0] for t in _AB], _f32)
    b = jnp.array([t[1] for t in _AB], _f32)
    c0 = jnp.zeros((_M, _DEPTH + 1), _f32).at[:, 0].set(1.0)
    al0 = alphas[0]
    c1 = (
        jnp.zeros((_M, _DEPTH + 1), _f32)
        .at[:, 0].set(al0 * (a - b) / 2.0)
        .at[:, 1].set(al0 * (a + b + 2.0) / 2.0)
    )
    cs = [c0, c1]
    for L in range(2, _DEPTH + 1):
        coef_l = 2.0 * L * (L + a + b) * (2.0 * L - 2.0 + a + b)
        coef_lm1_1 = (2.0 * L + a + b - 1.0) * (2.0 * L + a + b) * (2.0 * L + a + b - 2.0)
        coef_lm1_2 = (2.0 * L + a + b - 1.0) * (a * a - b * b)
        coef_lm2 = 2.0 * (L - 1.0 + a) * (L - 1.0 + b) * (2.0 * L + a + b)
        tmp1 = alphas[L - 1] * (coef_lm1_1 / coef_l)
        tmp2 = alphas[L - 1] * (coef_lm1_2 / coef_l)
        tmp3 = alphas[L - 1] * alphas[L - 2] * (coef_lm2 / coef_l)
        cp, cpp = cs[L - 1], cs[L - 2]
        shifted = jnp.concatenate([jnp.zeros((_M, 1), _f32), cp[:, :_DEPTH]], axis=1)
        cs.append(tmp1[:, None] * shifted - tmp2[:, None] * cp - tmp3[:, None] * cpp)
    wm = w.reshape(_M)
    return jnp.stack([(wm[:, None] * cL).sum(0) for cL in cs]).reshape(-1)


def _rsqrt16(d):
    i = lax.bitcast_convert_type(d, _i32)
    i = jnp.int32(0x5F3759DF) - lax.shift_right_arithmetic(i, jnp.int32(1))
    y = lax.bitcast_convert_type(i, _f32)
    for _ in range(3):
        y = y * (1.5 - 0.5 * d * y * y)
    return y


def _sc_body(xt_r, rowt_r, colt_r, attrt_r, cmat_r,
             out_r, y1_r, y2_r, y3_r,
             row_v, col_v, val_v, dinv_v, gbufA, gbufB,
             cvec, deg_sh, acc_sh, semA, semB, semSA, semSB):
    cid = lax.axis_index("c")
    sid = lax.axis_index("s")
    z16 = jnp.zeros((16,), _f32)
    one16 = jnp.ones((16,), _f32)

    pltpu.sync_copy(rowt_r.at[sid], row_v)
    pltpu.sync_copy(colt_r.at[sid], col_v)
    pltpu.sync_copy(attrt_r.at[sid], val_v)
    pltpu.sync_copy(cmat_r, cvec)

    @pl.loop(0, _BE)
    def _zero_rows0(r):
        for q in range(_DH // 16):
            gbufA[r, pl.ds(q * 16, 16)] = z16
    for k in range(_RT // _BE):
        pltpu.sync_copy(gbufA, acc_sh.at[pl.ds(sid * _RT + k * _BE, _BE)])

    @pl.loop(0, _BE)
    def _fill_ones(r):
        for q in range(_DH // 16):
            gbufA[r, pl.ds(q * 16, 16)] = one16
    plsc.subcore_barrier()

    @pl.loop(0, _NB // 2)
    def _hist(p):
        b = p * 2
        pltpu.async_copy(gbufA, acc_sh.at[row_v.at[b]], semSA, add=True)
        pltpu.async_copy(gbufA, acc_sh.at[row_v.at[b + 1]], semSB, add=True)
        pltpu.make_async_copy(gbufA, acc_sh.at[pl.ds(0, _BE)], semSA).wait()
        pltpu.make_async_copy(gbufA, acc_sh.at[pl.ds(0, _BE)], semSB).wait()
    plsc.subcore_barrier()

    iota16 = lax.iota(_i32, 16)
    for k in range(_RT // _BE):
        pltpu.sync_copy(acc_sh.at[pl.ds(sid * _RT + k * _BE, _BE)], gbufA)
        for g in range(_BE // 16):
            d = plsc.load_gather(gbufA, [iota16 + g * 16, jnp.zeros((16,), _i32)])
            d = d + jnp.where(d < 0.5, 1.0, 0.0).astype(_f32)
            dinv_v[pl.ds(k * _BE + g * 16, 16)] = _rsqrt16(d)
    pltpu.sync_copy(dinv_v.at[pl.ds(0, _RT)], deg_sh.at[pl.ds(sid * _RT, _RT)])
    plsc.subcore_barrier()

    pltpu.sync_copy(deg_sh, dinv_v)

    @pl.loop(0, _NB)
    def _val(bi):
        for g in range(_BE // 16):
            sl = pl.ds(g * 16, 16)
            r16 = row_v[bi, sl]
            c16 = col_v[bi, sl]
            dr = plsc.load_gather(dinv_v, [r16])
            dc = plsc.load_gather(dinv_v, [c16])
            val_v[bi, sl] = dr * val_v[bi, sl] * dc

    off16 = jnp.full((16,), cid * _NPAD, _i32)

    @pl.loop(0, _NB)
    def _off(bi):
        for g in range(_BE // 16):
            sl = pl.ds(g * 16, 16)
            col_v[bi, sl] = col_v[bi, sl] + off16

    def _scale(buf, bi):
        b16 = jnp.full((16,), bi, _i32)

        @pl.loop(0, _BE // 16)
        def _grp(g):
            e0 = g * 16
            for i in range(16):
                e = e0 + i
                vb = plsc.load_gather(val_v, [b16, jnp.full((16,), e, _i32)])
                for q in range(_DH // 16):
                    sl = pl.ds(q * 16, 16)
                    buf[e, sl] = buf[e, sl] * vb

    for src, dst in ((xt_r, y1_r), (y1_r, y2_r), (y2_r, y3_r)):
        @pl.loop(0, _BE)
        def _zero_rows(r):
            for q in range(_DH // 16):
                gbufA[r, pl.ds(q * 16, 16)] = z16
        for k in range(_RT // _BE):
            pltpu.sync_copy(gbufA, acc_sh.at[pl.ds(sid * _RT + k * _BE, _BE)])
        plsc.subcore_barrier()

        pltpu.async_copy(src.at[col_v.at[0]], gbufA, semA)

        @pl.loop(0, _NB // 2)
        def _pairs(p, src=src):
            b = p * 2
            pltpu.async_copy(src.at[col_v.at[b + 1]], gbufB, semB)
            pltpu.make_async_copy(src.at[pl.ds(0, _BE)], gbufA, semA).wait()
            _scale(gbufA, b)
            pltpu.async_copy(gbufA, acc_sh.at[row_v.at[b]], semSA, add=True)
            pltpu.make_async_copy(src.at[pl.ds(0, _BE)], gbufB, semB).wait()
            _scale(gbufB, b + 1)
            pltpu.async_copy(gbufB, acc_sh.at[row_v.at[b + 1]], semSB, add=True)
            pltpu.make_async_copy(gbufA, acc_sh.at[pl.ds(0, _BE)], semSA).wait()

            @pl.when(b + 2 < _NB)
            def _fire_next():
                pltpu.async_copy(src.at[col_v.at[b + 2]], gbufA, semA)

            pltpu.make_async_copy(gbufB, acc_sh.at[pl.ds(0, _BE)], semSB).wait()

        plsc.subcore_barrier()
        for k in range(_RT // _BE):
            r0 = sid * _RT + k * _BE
            pltpu.sync_copy(acc_sh.at[pl.ds(r0, _BE)], dst.at[pl.ds(cid * _NPAD + r0, _BE)])
        plsc.subcore_barrier()

    cb = [[cvec[4 * L + j, pl.ds(0, 16)] for j in range(4)] for L in range(4)]
    srcs = (xt_r, y1_r, y2_r, y3_r)

    @pl.loop(0, _RT // 32)
    def _combine(k):
        r0 = sid * _RT + k * 32
        for j in range(4):
            pltpu.async_copy(srcs[j].at[pl.ds(cid * _NPAD + r0, 32)],
                             gbufA.at[pl.ds(32 * j, 32)], semA)
        for j in range(4):
            pltpu.make_async_copy(srcs[0].at[pl.ds(0, 32)],
                                  gbufA.at[pl.ds(0, 32)], semA).wait()

        @pl.loop(0, 32)
        def _rows(r):
            for q in range(_DH // 16):
                sl = pl.ds(q * 16, 16)
                a = [gbufA[32 * j + r, sl] for j in range(4)]
                for L in range(4):
                    gbufB[r * 4 + L, sl] = (
                        a[0] * cb[L][0] + a[1] * cb[L][1]
                        + a[2] * cb[L][2] + a[3] * cb[L][3]
                    )

        pltpu.sync_copy(gbufB, out_r.at[cid, pl.ds(r0 * 4, _BE)])


@jax.jit
def kernel(x, edge_index, edge_attr, alphas, w):
    cmat = jnp.tile(_coef_matrix(alphas, w).reshape(16, 1), (1, 16))

    xp = jnp.pad(x, ((0, _NPAD - _N), (0, 0)))
    xt = xp.reshape(_NPAD, 2, _DH).transpose(1, 0, 2).reshape(2 * _NPAD, _DH)

    row = jnp.pad(edge_index[0].reshape(_NT, _ET), ((0, 0), (0, _ETP - _ET)),
                  constant_values=_N).reshape(_NT, _NB, _BE)
    col = jnp.pad(edge_index[1].reshape(_NT, _ET), ((0, 0), (0, _ETP - _ET)),
                  constant_values=0).reshape(_NT, _NB, _BE)
    attr = jnp.pad(edge_attr.reshape(_NT, _ET), ((0, 0), (0, _ETP - _ET)),
                   constant_values=0.0).reshape(_NT, _NB, _BE)

    mesh = plsc.VectorSubcoreMesh(core_axis_name="c", subcore_axis_name="s")
    fn = pl.kernel(
        _sc_body,
        out_type=(
            jax.ShapeDtypeStruct((2, _NPAD * (_DEPTH + 1), _DH), _f32),
            jax.ShapeDtypeStruct((2 * _NPAD, _DH), _f32),
            jax.ShapeDtypeStruct((2 * _NPAD, _DH), _f32),
            jax.ShapeDtypeStruct((2 * _NPAD, _DH), _f32),
        ),
        mesh=mesh,
        compiler_params=pltpu.CompilerParams(
            needs_layout_passes=False, use_tc_tiling_on_sc=False),
        scratch_types=[
            pltpu.VMEM((_NB, _BE), _i32),
            pltpu.VMEM((_NB, _BE), _i32),
            pltpu.VMEM((_NB, _BE), _f32),
            pltpu.VMEM((_NPAD,), _f32),
            pltpu.VMEM((_BE, _DH), _f32),
            pltpu.VMEM((_BE, _DH), _f32),
            pltpu.VMEM((16, 16), _f32),
            pltpu.VMEM_SHARED((_NPAD,), _f32),
            pltpu.VMEM_SHARED((_NPAD, _DH), _f32),
            pltpu.SemaphoreType.DMA,
            pltpu.SemaphoreType.DMA,
            pltpu.SemaphoreType.DMA,
            pltpu.SemaphoreType.DMA,
        ],
    )
    out_t, _, _, _ = fn(xt, row, col, attr, cmat)
    return (out_t.reshape(2, _NPAD, _DEPTH + 1, _DH)[:, :_N]
            .transpose(1, 2, 0, 3).reshape(_N, _DEPTH + 1, _D))


# speedup vs baseline: 1.7956x; 1.4757x over previous
import jax
import jax.numpy as jnp
import numpy as np
from jax import lax
from jax.experimental import pallas as pl
from jax.experimental.pallas import tpu as pltpu
from jax.experimental.pallas import tpu_sc as plsc

_DEPTH = 3
_STEP = 0.5
_AB = [(0.0, float(b)) for b in np.arange(0, 3, _STEP)] + [
    (float(a), 0.0) for a in np.arange(_STEP, 3, _STEP)
]
_M = len(_AB)
_N = 10000
_E = 320000
_D = 128
_DH = _D // 2

_NT = 16
_NPAD = 10240
_RT = _NPAD // _NT
_ET = _E // _NT
_BE = 128
_NB = 158
_ETP = _NB * _BE

_f32 = jnp.float32
_i32 = jnp.int32


def _coef_matrix(alphas, w):
    a = jnp.array([t[0] for t in _AB], _f32)
    b = jnp.array([t[1] for t in _AB], _f32)
    c0 = jnp.zeros((_M, _DEPTH + 1), _f32).at[:, 0].set(1.0)
    al0 = alphas[0]
    c1 = (
        jnp.zeros((_M, _DEPTH + 1), _f32)
        .at[:, 0].set(al0 * (a - b) / 2.0)
        .at[:, 1].set(al0 * (a + b + 2.0) / 2.0)
    )
    cs = [c0, c1]
    for L in range(2, _DEPTH + 1):
        coef_l = 2.0 * L * (L + a + b) * (2.0 * L - 2.0 + a + b)
        coef_lm1_1 = (2.0 * L + a + b - 1.0) * (2.0 * L + a + b) * (2.0 * L + a + b - 2.0)
        coef_lm1_2 = (2.0 * L + a + b - 1.0) * (a * a - b * b)
        coef_lm2 = 2.0 * (L - 1.0 + a) * (L - 1.0 + b) * (2.0 * L + a + b)
        tmp1 = alphas[L - 1] * (coef_lm1_1 / coef_l)
        tmp2 = alphas[L - 1] * (coef_lm1_2 / coef_l)
        tmp3 = alphas[L - 1] * alphas[L - 2] * (coef_lm2 / coef_l)
        cp, cpp = cs[L - 1], cs[L - 2]
        shifted = jnp.concatenate([jnp.zeros((_M, 1), _f32), cp[:, :_DEPTH]], axis=1)
        cs.append(tmp1[:, None] * shifted - tmp2[:, None] * cp - tmp3[:, None] * cpp)
    wm = w.reshape(_M)
    return jnp.stack([(wm[:, None] * cL).sum(0) for cL in cs]).reshape(-1)


def _rsqrt16(d):
    i = lax.bitcast_convert_type(d, _i32)
    i = jnp.int32(0x5F3759DF) - lax.shift_right_arithmetic(i, jnp.int32(1))
    y = lax.bitcast_convert_type(i, _f32)
    for _ in range(3):
        y = y * (1.5 - 0.5 * d * y * y)
    return y


def _sc_body(xt_r, rowt_r, colt_r, attrt_r, cmat_r,
             out_r, y1_r, y2_r, y3_r,
             row_v, col_v, val_v, dinv_v, gbufA, gbufB,
             cvec, deg_sh, acc_sh, semA, semB):
    cid = lax.axis_index("c")
    sid = lax.axis_index("s")
    z16 = jnp.zeros((16,), _f32)
    one16 = jnp.ones((16,), _f32)

    pltpu.sync_copy(rowt_r.at[sid], row_v)
    pltpu.sync_copy(colt_r.at[sid], col_v)
    pltpu.sync_copy(attrt_r.at[sid], val_v)
    pltpu.sync_copy(cmat_r, cvec)

    @pl.loop(0, _BE)
    def _zero_rows0(r):
        for q in range(_DH // 16):
            gbufA[r, pl.ds(q * 16, 16)] = z16
    for k in range(_RT // _BE):
        pltpu.sync_copy(gbufA, acc_sh.at[pl.ds(sid * _RT + k * _BE, _BE)])

    @pl.loop(0, _BE)
    def _fill_ones(r):
        for q in range(_DH // 16):
            gbufA[r, pl.ds(q * 16, 16)] = one16
    plsc.subcore_barrier()

    @pl.loop(0, _NB)
    def _hist(bi):
        pltpu.sync_copy(gbufA, acc_sh.at[row_v.at[bi]], add=True)
    plsc.subcore_barrier()

    iota16 = lax.iota(_i32, 16)
    for k in range(_RT // _BE):
        pltpu.sync_copy(acc_sh.at[pl.ds(sid * _RT + k * _BE, _BE)], gbufA)
        for g in range(_BE // 16):
            d = plsc.load_gather(gbufA, [iota16 + g * 16, jnp.zeros((16,), _i32)])
            d = d + jnp.where(d < 0.5, 1.0, 0.0).astype(_f32)
            dinv_v[pl.ds(k * _BE + g * 16, 16)] = _rsqrt16(d)
    pltpu.sync_copy(dinv_v.at[pl.ds(0, _RT)], deg_sh.at[pl.ds(sid * _RT, _RT)])
    plsc.subcore_barrier()

    pltpu.sync_copy(deg_sh, dinv_v)

    @pl.loop(0, _NB)
    def _val(bi):
        for g in range(_BE // 16):
            sl = pl.ds(g * 16, 16)
            r16 = row_v[bi, sl]
            c16 = col_v[bi, sl]
            dr = plsc.load_gather(dinv_v, [r16])
            dc = plsc.load_gather(dinv_v, [c16])
            val_v[bi, sl] = dr * val_v[bi, sl] * dc

    off16 = jnp.full((16,), cid * _NPAD, _i32)

    @pl.loop(0, _NB)
    def _off(bi):
        for g in range(_BE // 16):
            sl = pl.ds(g * 16, 16)
            col_v[bi, sl] = col_v[bi, sl] + off16

    def _scale(buf, bi):
        b16 = jnp.full((16,), bi, _i32)

        @plsc.parallel_loop(0, _BE, step=16, unroll=2)
        def _grp(e0):
            for i in range(16):
                e = e0 + i
                vb = plsc.load_gather(val_v, [b16, jnp.full((16,), e, _i32)])
                for q in range(_DH // 16):
                    sl = pl.ds(q * 16, 16)
                    buf[e, sl] = buf[e, sl] * vb

    for src, dst in ((xt_r, y1_r), (y1_r, y2_r), (y2_r, y3_r)):
        @pl.loop(0, _BE)
        def _zero_rows(r):
            for q in range(_DH // 16):
                gbufA[r, pl.ds(q * 16, 16)] = z16
        for k in range(_RT // _BE):
            pltpu.sync_copy(gbufA, acc_sh.at[pl.ds(sid * _RT + k * _BE, _BE)])
        plsc.subcore_barrier()

        pltpu.async_copy(src.at[col_v.at[0]], gbufA, semA)

        @pl.loop(0, _NB // 2)
        def _pairs(p, src=src):
            b = p * 2
            pltpu.async_copy(src.at[col_v.at[b + 1]], gbufB, semB)
            pltpu.make_async_copy(src.at[pl.ds(0, _BE)], gbufA, semA).wait()
            _scale(gbufA, b)
            pltpu.sync_copy(gbufA, acc_sh.at[row_v.at[b]], add=True)

            @pl.when(b + 2 < _NB)
            def _fire_next():
                pltpu.async_copy(src.at[col_v.at[b + 2]], gbufA, semA)

            pltpu.make_async_copy(src.at[pl.ds(0, _BE)], gbufB, semB).wait()
            _scale(gbufB, b + 1)
            pltpu.sync_copy(gbufB, acc_sh.at[row_v.at[b + 1]], add=True)

        plsc.subcore_barrier()
        for k in range(_RT // _BE):
            r0 = sid * _RT + k * _BE
            pltpu.sync_copy(acc_sh.at[pl.ds(r0, _BE)], dst.at[pl.ds(cid * _NPAD + r0, _BE)])
        plsc.subcore_barrier()

    cb = [[cvec[4 * L + j, pl.ds(0, 16)] for j in range(4)] for L in range(4)]
    srcs = (xt_r, y1_r, y2_r, y3_r)

    @pl.loop(0, _RT // 32)
    def _combine(k):
        r0 = sid * _RT + k * 32
        for j in range(4):
            pltpu.async_copy(srcs[j].at[pl.ds(cid * _NPAD + r0, 32)],
                             gbufA.at[pl.ds(32 * j, 32)], semA)
        for j in range(4):
            pltpu.make_async_copy(srcs[0].at[pl.ds(0, 32)],
                                  gbufA.at[pl.ds(0, 32)], semA).wait()

        @pl.loop(0, 32)
        def _rows(r):
            for q in range(_DH // 16):
                sl = pl.ds(q * 16, 16)
                a = [gbufA[32 * j + r, sl] for j in range(4)]
                for L in range(4):
                    gbufB[r * 4 + L, sl] = (
                        a[0] * cb[L][0] + a[1] * cb[L][1]
                        + a[2] * cb[L][2] + a[3] * cb[L][3]
                    )

        pltpu.sync_copy(gbufB, out_r.at[cid, pl.ds(r0 * 4, _BE)])


@jax.jit
def kernel(x, edge_index, edge_attr, alphas, w):
    cmat = jnp.tile(_coef_matrix(alphas, w).reshape(16, 1), (1, 16))

    xp = jnp.pad(x, ((0, _NPAD - _N), (0, 0)))
    xt = xp.reshape(_NPAD, 2, _DH).transpose(1, 0, 2).reshape(2 * _NPAD, _DH)

    row = jnp.pad(edge_index[0].reshape(_NT, _ET), ((0, 0), (0, _ETP - _ET)),
                  constant_values=_N).reshape(_NT, _NB, _BE)
    col = jnp.pad(edge_index[1].reshape(_NT, _ET), ((0, 0), (0, _ETP - _ET)),
                  constant_values=0).reshape(_NT, _NB, _BE)
    attr = jnp.pad(edge_attr.reshape(_NT, _ET), ((0, 0), (0, _ETP - _ET)),
                   constant_values=0.0).reshape(_NT, _NB, _BE)

    mesh = plsc.VectorSubcoreMesh(core_axis_name="c", subcore_axis_name="s")
    fn = pl.kernel(
        _sc_body,
        out_type=(
            jax.ShapeDtypeStruct((2, _NPAD * (_DEPTH + 1), _DH), _f32),
            jax.ShapeDtypeStruct((2 * _NPAD, _DH), _f32),
            jax.ShapeDtypeStruct((2 * _NPAD, _DH), _f32),
            jax.ShapeDtypeStruct((2 * _NPAD, _DH), _f32),
        ),
        mesh=mesh,
        compiler_params=pltpu.CompilerParams(
            needs_layout_passes=False, use_tc_tiling_on_sc=False),
        scratch_types=[
            pltpu.VMEM((_NB, _BE), _i32),
            pltpu.VMEM((_NB, _BE), _i32),
            pltpu.VMEM((_NB, _BE), _f32),
            pltpu.VMEM((_NPAD,), _f32),
            pltpu.VMEM((_BE, _DH), _f32),
            pltpu.VMEM((_BE, _DH), _f32),
            pltpu.VMEM((16, 16), _f32),
            pltpu.VMEM_SHARED((_NPAD,), _f32),
            pltpu.VMEM_SHARED((_NPAD, _DH), _f32),
            pltpu.SemaphoreType.DMA,
            pltpu.SemaphoreType.DMA,
        ],
    )
    out_t, _, _, _ = fn(xt, row, col, attr, cmat)
    return (out_t.reshape(2, _NPAD, _DEPTH + 1, _DH)[:, :_N]
            .transpose(1, 2, 0, 3).reshape(_N, _DEPTH + 1, _D))
